# baseline (speedup 1.0000x reference)
import functools

import jax
import jax.numpy as jnp
from jax import lax
from jax.experimental import pallas as pl
from jax.experimental.pallas import tpu as pltpu
from jax.experimental.pallas import tpu_sc as plsc

_B = 16384
_L = 9
_K = 10
_ZROW = 1000001
_TROWS = 125008
_NCORES = 2
_NSUB = 16
_NW = _NCORES * _NSUB
_BC = _B // _NW
_NCH = _BC * _L // 128
_NCHP = 40
_UC = 4
_UCP = 8

_mesh = plsc.VectorSubcoreMesh(
    core_axis_name="c", subcore_axis_name="s",
    num_cores=_NCORES, num_subcores=_NSUB,
)


@functools.partial(
    pl.kernel,
    out_type=(
        jax.ShapeDtypeStruct((_L, _B, 128), jnp.float32),
        jax.ShapeDtypeStruct((_B, 128), jnp.float32),
    ),
    mesh=_mesh,
    scratch_types=[
        pltpu.VMEM((_NCHP, 128), jnp.int32),
        pltpu.VMEM((_UCP, 128), jnp.int32),
        pltpu.VMEM((2, 128, 128), jnp.float32),
        pltpu.SemaphoreType.DMA,
        pltpu.SemaphoreType.DMA,
        pltpu.SemaphoreType.DMA,
        pltpu.SemaphoreType.DMA,
    ],
    compiler_params=pltpu.CompilerParams(use_tc_tiling_on_sc=True),
)
def _sc_gather(nbq_hbm, uq_hbm, table_hbm, nbslab_hbm, uslab_hbm,
               nbq_v, uq_v, buf_v, sem_g0, sem_g1, sem_w0, sem_w1):
    wid = lax.axis_index("s") * _NCORES + lax.axis_index("c")
    base = wid * _BC

    pltpu.sync_copy(nbq_hbm.at[pl.ds(wid * _NCHP, _NCHP)], nbq_v)
    pltpu.sync_copy(uq_hbm.at[pl.ds(wid * _UCP, _UCP)], uq_v)

    sem_g = (sem_g0, sem_g1)
    sem_w = (sem_w0, sem_w1)

    def _nb_dst(c):
        return nbslab_hbm.at[c // 4, pl.ds(base + (c % 4) * 128, 128)]

    def _fire(c, p):
        pltpu.async_copy(table_hbm.at[nbq_v.at[c]], buf_v.at[p], sem_g[p])

    def _wait_g(c, p):
        pltpu.make_async_copy(table_hbm.at[nbq_v.at[c]], buf_v.at[p],
                              sem_g[p]).wait()

    def _fire_w(c, p):
        pltpu.async_copy(buf_v.at[p], _nb_dst(c), sem_w[p])

    def _wait_w(c, p):
        pltpu.make_async_copy(buf_v.at[p], _nb_dst(c), sem_w[p]).wait()

    _fire(0, 0)

    def _pair(t, carry):
        c0 = 2 * t
        _wait_g(c0, 0)

        @pl.when(t > 0)
        def _():
            _wait_w(c0 - 1, 1)
        _fire(c0 + 1, 1)
        _fire_w(c0, 0)

        _wait_g(c0 + 1, 1)

        @pl.when(c0 + 2 < _NCH)
        def _():
            _wait_w(c0, 0)
            _fire(c0 + 2, 0)
        _fire_w(c0 + 1, 1)
        return carry
    lax.fori_loop(0, _NCH // 2, _pair, 0)
    _wait_w(_NCH - 2, 0)
    _wait_w(_NCH - 1, 1)

    for c in range(_UC):
        pltpu.async_copy(table_hbm.at[uq_v.at[c]], buf_v.at[0],
                         sem_g0).wait()
        pltpu.sync_copy(buf_v.at[0],
                        uslab_hbm.at[pl.ds(base + c * 128, 128)])


_BS = 256


def _tc_body(offt_ref, uoff_ref, nbslab_ref, uslab_ref, out_ref):
    lane = lax.broadcasted_iota(jnp.int32, (_BS, 128), 1) >> 4
    acc = jnp.zeros((_BS, 128), jnp.float32)
    for l in range(_L):
        m = lane == offt_ref[l, :][:, None]
        acc = acc + jnp.where(m, nbslab_ref[l], 0.0)
    um = lane == uoff_ref[0, 0, :][:, None]
    u = jnp.where(um, uslab_ref[...], 0.0)

    krow = lax.broadcasted_iota(jnp.int32, (128, 16), 0)
    kcol = lax.broadcasted_iota(jnp.int32, (128, 16), 1)
    fold = (krow % 16 == kcol).astype(jnp.float32)
    nb16 = jnp.dot(acc, fold, preferred_element_type=jnp.float32,
                   precision=lax.Precision.HIGHEST)
    u16 = jnp.dot(u, fold, preferred_element_type=jnp.float32,
                  precision=lax.Precision.HIGHEST)
    out_ref[0, 0, :] = jnp.sum(u16 * nb16, axis=1)


_tc_finish = pl.pallas_call(
    _tc_body,
    grid=(_B // _BS,),
    in_specs=[
        pl.BlockSpec((_L, _BS), lambda i: (0, i)),
        pl.BlockSpec((1, 1, _BS), lambda i: (i, 0, 0)),
        pl.BlockSpec((_L, _BS, 128), lambda i: (0, i, 0)),
        pl.BlockSpec((_BS, 128), lambda i: (i, 0)),
    ],
    out_specs=pl.BlockSpec((1, 1, _BS), lambda i: (i, 0, 0)),
    out_shape=jax.ShapeDtypeStruct((_B // _BS, 1, _BS), jnp.float32),
)


@jax.jit
def kernel(user, nearby, table):
    t128 = jnp.pad(table, ((0, 63), (0, 6))).reshape(_TROWS, 128)

    u32 = user.astype(jnp.int32)
    nb32 = nearby.astype(jnp.int32)
    u32 = jnp.where(u32 == 0, _ZROW, u32)
    nb32 = jnp.where(nb32 == 0, _ZROW, nb32)

    nbt = nb32.T
    nbq = (nbt >> 3).reshape(_L, _NW, _BC).transpose(1, 0, 2).reshape(
        _NW, _NCH, 128)
    nbq = jnp.pad(nbq, ((0, 0), (0, _NCHP - _NCH), (0, 0)))
    nbq = nbq.reshape(_NW * _NCHP, 128)

    uq = (u32 >> 3).reshape(_NW, _UC, 128)
    uq = jnp.pad(uq, ((0, 0), (0, _UCP - _UC), (0, 0)))
    uq = uq.reshape(_NW * _UCP, 128)

    nbslab, uslab = _sc_gather(nbq, uq, t128)

    offt = nbt & 7
    uoff = (u32 & 7).reshape(_B // _BS, 1, _BS)
    preds = _tc_finish(offt, uoff, nbslab, uslab)
    return preds.reshape(_B)


# speedup vs baseline: 1.1257x; 1.1257x over previous
import functools

import jax
import jax.numpy as jnp
from jax import lax
from jax.experimental import pallas as pl
from jax.experimental.pallas import tpu as pltpu
from jax.experimental.pallas import tpu_sc as plsc

_B = 16384
_L = 9
_K = 10
_KP = 16
_NROWS = 1000064
_ZROW = 1000001
_NCORES = 2
_NSUB = 16
_NW = _NCORES * _NSUB
_BC = _B // _NW
_NBC = _BC * _L
_CH = 128
_UC = _BC // _CH
_NCH = _NBC // _CH
_UCP = 8
_NCHP = 40
_PR = _B * _KP // 128
_PRT = _PR // _NW

_mesh = plsc.VectorSubcoreMesh(
    core_axis_name="c", subcore_axis_name="s",
    num_cores=_NCORES, num_subcores=_NSUB,
)


@functools.partial(
    pl.kernel,
    out_type=jax.ShapeDtypeStruct((_PR, 128), jnp.float32),
    mesh=_mesh,
    scratch_types=[
        pltpu.VMEM((_UCP, _CH), jnp.int32),
        pltpu.VMEM((_NCHP, _CH), jnp.int32),
        pltpu.VMEM((_BC, _KP), jnp.float32),
        pltpu.VMEM((_NBC, _KP), jnp.float32),
        pltpu.VMEM((_PRT, 128), jnp.float32),
        pltpu.SemaphoreType.DMA,
        pltpu.SemaphoreType.DMA,
    ],
    compiler_params=pltpu.CompilerParams(use_tc_tiling_on_sc=False),
)
def _place_sc(user_hbm, nearby_hbm, table_hbm, psum_hbm,
              uidx_v, nbidx_v, urows_v, nbrows_v, psum_v, sem_u, sem_nb):
    wid = lax.axis_index("s") * _NCORES + lax.axis_index("c")

    pltpu.sync_copy(user_hbm.at[pl.ds(wid * _UCP, _UCP)], uidx_v)
    pltpu.sync_copy(nearby_hbm.at[pl.ds(wid * _NCHP, _NCHP)], nbidx_v)

    for c in range(_UC):
        pltpu.async_copy(table_hbm.at[uidx_v.at[c]],
                         urows_v.at[pl.ds(c * _CH, _CH)], sem_u).wait()

    def _fire(c, carry):
        pltpu.async_copy(table_hbm.at[nbidx_v.at[c]],
                         nbrows_v.at[pl.ds(c * _CH, _CH)], sem_nb).wait()
        return carry
    lax.fori_loop(0, _NCH, _fire, 0)

    def _one(b, carry):
        s = nbrows_v[b * _L, :]
        for l in range(1, _L):
            s = s + nbrows_v[b * _L + l, :]
        p = s * urows_v[b, :]
        psum_v[b >> 3, pl.ds((b & 7) * _KP, _KP)] = p
        return carry
    lax.fori_loop(0, _BC, _one, 0)

    pltpu.sync_copy(psum_v, psum_hbm.at[pl.ds(wid * _PRT, _PRT)])


_BS = 256


def _tc_body(psum_ref, out_ref):
    rows = lax.broadcasted_iota(jnp.int32, (128, 8), 0)
    cols = lax.broadcasted_iota(jnp.int32, (128, 8), 1)
    bd = (rows // 16 == cols).astype(jnp.float32)
    out_ref[...] = jnp.dot(psum_ref[...], bd,
                           preferred_element_type=jnp.float32,
                           precision=lax.Precision.HIGHEST)


_tc_reduce = pl.pallas_call(
    _tc_body,
    grid=(_PR // _BS,),
    in_specs=[pl.BlockSpec((_BS, 128), lambda i: (i, 0))],
    out_specs=pl.BlockSpec((_BS, 8), lambda i: (i, 0)),
    out_shape=jax.ShapeDtypeStruct((_PR, 8), jnp.float32),
)


@jax.jit
def kernel(user, nearby, table):
    t16 = jnp.pad(table, ((0, _NROWS - table.shape[0]), (0, _KP - _K)))

    u32 = user.astype(jnp.int32)
    nb32 = nearby.astype(jnp.int32)
    u32 = jnp.where(u32 == 0, _ZROW, u32)
    nb32 = jnp.where(nb32 == 0, _ZROW, nb32)

    uidx = u32.reshape(_NW, _UC, _CH)
    uidx = jnp.pad(uidx, ((0, 0), (0, _UCP - _UC), (0, 0)))
    uidx = uidx.reshape(_NW * _UCP, _CH)

    nbidx = nb32.reshape(_NW, _NCH, _CH)
    nbidx = jnp.pad(nbidx, ((0, 0), (0, _NCHP - _NCH), (0, 0)))
    nbidx = nbidx.reshape(_NW * _NCHP, _CH)

    psum = _place_sc(uidx, nbidx, t16)
    return _tc_reduce(psum).reshape(_B)
